# baseline (speedup 1.0000x reference)
import functools

import jax
import jax.numpy as jnp
from jax.experimental import pallas as pl
from jax.experimental.pallas import tpu as pltpu

_E = 4
_K = 2
_D = 512
_H = 4 * _D


def _router_body(x_ref, wg_ref, c_ref):
    x = x_ref[...]
    logits = jnp.dot(x, wg_ref[...], preferred_element_type=jnp.float32)
    m = jnp.max(logits, axis=-1, keepdims=True)
    ex = jnp.exp(logits - m)
    p = ex / jnp.sum(ex, axis=-1, keepdims=True)
    col = jax.lax.broadcasted_iota(jnp.int32, p.shape, 1)
    i1 = jnp.argmax(p, axis=-1)[:, None]
    m1 = jnp.max(p, axis=-1, keepdims=True)
    p_wo = jnp.where(col == i1, -jnp.inf, p)
    i2 = jnp.argmax(p_wo, axis=-1)[:, None]
    m2 = jnp.max(p_wo, axis=-1, keepdims=True)
    denom = jnp.maximum(m1 + m2, 1e-6)
    sel = (col == i1) | (col == i2)
    c_ref[...] = jnp.where(sel, p / denom, 0.0)


def _moe_body(x_ref, w1_ref, w2_ref, c_ref, o_ref):
    e = pl.program_id(1)
    x = x_ref[...]
    h = jnp.dot(x, w1_ref[0], preferred_element_type=jnp.float32)
    h = 0.5 * h * (1.0 + jnp.tanh(0.7978845608028654 * (h + 0.044715 * h * h * h)))
    y = jnp.dot(h.astype(jnp.bfloat16), w2_ref[0], preferred_element_type=jnp.float32)
    c = c_ref[...]
    col = jax.lax.broadcasted_iota(jnp.int32, c.shape, 1)
    w = jnp.sum(jnp.where(col == e, c, 0.0), axis=1, keepdims=True)
    y = y * w

    @pl.when(e == 0)
    def _():
        o_ref[...] = y

    @pl.when(e != 0)
    def _():
        o_ref[...] += y


def kernel(x, Wg, W1, W2):
    b, t, c = x.shape
    n = b * t
    x_flat = x.reshape(n, c)

    router = pl.pallas_call(
        _router_body,
        out_shape=jax.ShapeDtypeStruct((n, _E), jnp.float32),
        grid=(1,),
        in_specs=[
            pl.BlockSpec((n, _D), lambda i: (0, 0)),
            pl.BlockSpec((_D, _E), lambda i: (0, 0)),
        ],
        out_specs=pl.BlockSpec((n, _E), lambda i: (0, 0)),
    )
    C = router(x_flat, Wg)

    rt = 512
    moe = pl.pallas_call(
        _moe_body,
        out_shape=jax.ShapeDtypeStruct((n, _D), jnp.float32),
        grid=(n // rt, _E),
        in_specs=[
            pl.BlockSpec((rt, _D), lambda i, e: (i, 0)),
            pl.BlockSpec((1, _D, _H), lambda i, e: (e, 0, 0)),
            pl.BlockSpec((1, _H, _D), lambda i, e: (e, 0, 0)),
            pl.BlockSpec((rt, _E), lambda i, e: (i, 0)),
        ],
        out_specs=pl.BlockSpec((rt, _D), lambda i, e: (i, 0)),
    )
    out_flat = moe(
        x_flat.astype(jnp.bfloat16),
        W1.astype(jnp.bfloat16),
        W2.astype(jnp.bfloat16),
        C,
    )
    return out_flat.reshape(b, t, c), C


# speedup vs baseline: 1.0869x; 1.0869x over previous
import functools

import jax
import jax.numpy as jnp
from jax.experimental import pallas as pl
from jax.experimental.pallas import tpu as pltpu

_E = 4
_K = 2
_D = 512
_H = 4 * _D


def _router_body(x_ref, wg_ref, c_ref):
    x = x_ref[...]
    logits = jnp.dot(x, wg_ref[...], preferred_element_type=jnp.float32)
    m = jnp.max(logits, axis=-1, keepdims=True)
    ex = jnp.exp(logits - m)
    p = ex / jnp.sum(ex, axis=-1, keepdims=True)
    col = jax.lax.broadcasted_iota(jnp.int32, p.shape, 1)
    i1 = jnp.argmax(p, axis=-1)[:, None]
    m1 = jnp.max(p, axis=-1, keepdims=True)
    p_wo = jnp.where(col == i1, -jnp.inf, p)
    i2 = jnp.argmax(p_wo, axis=-1)[:, None]
    m2 = jnp.max(p_wo, axis=-1, keepdims=True)
    denom = jnp.maximum(m1 + m2, 1e-6)
    sel = (col == i1) | (col == i2)
    c_ref[...] = jnp.where(sel, p / denom, 0.0)


def _moe_body(x_ref, w1_ref, w2_ref, c_ref, o_ref):
    x = x_ref[...]
    c = c_ref[...]
    col = jax.lax.broadcasted_iota(jnp.int32, c.shape, 1)
    acc = None
    for e in range(_E):
        h = jnp.dot(x, w1_ref[e], preferred_element_type=jnp.float32)
        h = 0.5 * h * (1.0 + jax.lax.erf(h * 0.7071067811865476))
        y = jnp.dot(h.astype(jnp.bfloat16), w2_ref[e],
                    preferred_element_type=jnp.float32)
        w = jnp.sum(jnp.where(col == e, c, 0.0), axis=1, keepdims=True)
        y = y * w
        acc = y if acc is None else acc + y
    o_ref[...] = acc


def kernel(x, Wg, W1, W2):
    b, t, c = x.shape
    n = b * t
    x_flat = x.reshape(n, c)

    router = pl.pallas_call(
        _router_body,
        out_shape=jax.ShapeDtypeStruct((n, _E), jnp.float32),
        grid=(1,),
        in_specs=[
            pl.BlockSpec((n, _D), lambda i: (0, 0)),
            pl.BlockSpec((_D, _E), lambda i: (0, 0)),
        ],
        out_specs=pl.BlockSpec((n, _E), lambda i: (0, 0)),
    )
    C = router(x_flat, Wg)

    rt = 512
    moe = pl.pallas_call(
        _moe_body,
        out_shape=jax.ShapeDtypeStruct((n, _D), jnp.float32),
        grid=(n // rt,),
        in_specs=[
            pl.BlockSpec((rt, _D), lambda i: (i, 0)),
            pl.BlockSpec((_E, _D, _H), lambda i: (0, 0, 0)),
            pl.BlockSpec((_E, _H, _D), lambda i: (0, 0, 0)),
            pl.BlockSpec((rt, _E), lambda i: (i, 0)),
        ],
        out_specs=pl.BlockSpec((rt, _D), lambda i: (i, 0)),
    )
    out_flat = moe(
        x_flat.astype(jnp.bfloat16),
        W1.astype(jnp.bfloat16),
        W2.astype(jnp.bfloat16),
        C,
    )
    return out_flat.reshape(b, t, c), C
